# Initial kernel scaffold; baseline (speedup 1.0000x reference)
import functools

import jax
import jax.numpy as jnp
from jax.experimental import pallas as pl

H = 768
I = 128
E = 16
SH_I = 256
T = 2048
TB = 128
NB = T // TB


def _dot_nt(a, b):
    return jax.lax.dot_general(a, b, (((1,), (1,)), ((), ())),
                               preferred_element_type=jnp.float32)


def _moe_block(x_ref, gate_w_ref, ew1_ref, eb1_ref, ew2_ref, eb2_ref,
               ew3_ref, eb3_ref, sw1_ref, sb1_ref, sw2_ref, sb2_ref,
               sw3_ref, sb3_ref, out_ref):
    xb = x_ref[...]

    scores = _dot_nt(xb, gate_w_ref[...])
    smax = jnp.max(scores, axis=-1, keepdims=True)
    p = jnp.exp(scores - smax)
    probs = p / jnp.sum(p, axis=-1, keepdims=True)
    pmax = jnp.max(probs, axis=-1, keepdims=True)
    lane = jax.lax.broadcasted_iota(jnp.int32, (TB, E), 1)
    first_max = jnp.min(jnp.where(probs >= pmax, lane, E), axis=-1,
                        keepdims=True)
    cw = jnp.where(lane == first_max, pmax, 0.0)

    t1 = _dot_nt(xb, sw1_ref[...]) + sb1_ref[...]
    t3 = _dot_nt(xb, sw3_ref[...]) + sb3_ref[...]
    acc = _dot_nt(jax.nn.silu(t1) * t3, sw2_ref[...].T) + sb2_ref[...]

    for e in range(E):
        h1 = _dot_nt(xb, ew1_ref[e]) + eb1_ref[e]
        h3 = _dot_nt(xb, ew3_ref[e]) + eb3_ref[e]
        h = jax.nn.silu(h1) * h3
        oe = _dot_nt(h, ew2_ref[e]) + eb2_ref[e]
        acc = acc + cw[:, e:e + 1] * oe

    out_ref[...] = acc


@functools.partial(jax.jit, static_argnames=("interpret",))
def _run(x, gate_w, ew1, eb1, ew2, eb2, ew3, eb3, sw1, sb1, sw2, sb2,
         sw3, sb3, interpret=False):
    full = lambda shape: pl.BlockSpec(shape, lambda b: (0,) * len(shape))
    return pl.pallas_call(
        _moe_block,
        grid=(NB,),
        in_specs=[
            pl.BlockSpec((TB, H), lambda b: (b, 0)),
            full((E, H)),
            full((E, I, H)), full((E, 1, I)),
            full((E, H, I)), full((E, 1, H)),
            full((E, I, H)), full((E, 1, I)),
            full((SH_I, H)), full((1, SH_I)),
            full((H, SH_I)), full((1, H)),
            full((SH_I, H)), full((1, SH_I)),
        ],
        out_specs=pl.BlockSpec((TB, H), lambda b: (b, 0)),
        out_shape=jax.ShapeDtypeStruct((T, H), jnp.float32),
        interpret=interpret,
    )(x, gate_w, ew1, eb1, ew2, eb2, ew3, eb3, sw1, sb1, sw2, sb2, sw3, sb3)


def kernel(hidden_states, gate_w, ew1, eb1, ew2, eb2, ew3, eb3, sw1, sb1,
           sw2, sb2, sw3, sb3):
    shape = hidden_states.shape
    x = hidden_states.reshape(-1, H)
    y = _run(x, gate_w,
             ew1, eb1.reshape(E, 1, I), ew2, eb2.reshape(E, 1, H),
             ew3, eb3.reshape(E, 1, I),
             sw1, sb1.reshape(1, SH_I), sw2, sb2.reshape(1, H),
             sw3, sb3.reshape(1, SH_I))
    return y.reshape(shape)


# speedup vs baseline: 1.8144x; 1.8144x over previous
import functools

import jax
import jax.numpy as jnp
from jax.experimental import pallas as pl

H = 768
I = 128
E = 16
SH_I = 256
T = 2048
TB = 128
NB = T // TB


def _dot_nt(a, b):
    return jax.lax.dot_general(a, b, (((1,), (1,)), ((), ())),
                               preferred_element_type=jnp.float32)


def _moe_block(x_ref, gate_w_ref, ew1_ref, eb1_ref, ew2_ref, eb2_ref,
               ew3_ref, eb3_ref, sw1_ref, sb1_ref, sw2_ref, sb2_ref,
               sw3_ref, sb3_ref, out_ref):
    xb = x_ref[...]

    scores = _dot_nt(xb, gate_w_ref[...])
    smax = jnp.max(scores, axis=-1, keepdims=True)
    p = jnp.exp(scores - smax)
    probs = p / jnp.sum(p, axis=-1, keepdims=True)
    pmax = jnp.max(probs, axis=-1, keepdims=True)
    lane = jax.lax.broadcasted_iota(jnp.int32, (TB, E), 1)
    first_max = jnp.min(jnp.where(probs >= pmax, lane, E), axis=-1,
                        keepdims=True)
    cw = jnp.where(lane == first_max, pmax, 0.0)

    t1 = _dot_nt(xb, sw1_ref[...]) + sb1_ref[...]
    t3 = _dot_nt(xb, sw3_ref[...]) + sb3_ref[...]
    acc = _dot_nt(jax.nn.silu(t1) * t3, sw2_ref[...]) + sb2_ref[...]

    for e in range(E):
        h1 = _dot_nt(xb, ew1_ref[e]) + eb1_ref[e]
        h3 = _dot_nt(xb, ew3_ref[e]) + eb3_ref[e]
        h = jax.nn.silu(h1) * h3
        oe = _dot_nt(h, ew2_ref[e]) + eb2_ref[e]
        acc = acc + cw[:, e:e + 1] * oe

    out_ref[...] = acc


@functools.partial(jax.jit, static_argnames=("interpret",))
def _run(x, gate_w, ew1, eb1, ew2, eb2, ew3, eb3, sw1, sb1, sw2, sb2,
         sw3, sb3, interpret=False):
    full = lambda shape: pl.BlockSpec(shape, lambda b: (0,) * len(shape))
    return pl.pallas_call(
        _moe_block,
        grid=(NB,),
        in_specs=[
            pl.BlockSpec((TB, H), lambda b: (b, 0)),
            full((E, H)),
            full((E, I, H)), full((E, 1, I)),
            full((E, H, I)), full((E, 1, H)),
            full((E, I, H)), full((E, 1, I)),
            full((SH_I, H)), full((1, SH_I)),
            full((H, SH_I)), full((1, H)),
            full((SH_I, H)), full((1, SH_I)),
        ],
        out_specs=pl.BlockSpec((TB, H), lambda b: (b, 0)),
        out_shape=jax.ShapeDtypeStruct((T, H), jnp.float32),
        interpret=interpret,
    )(x, gate_w, ew1, eb1, ew2, eb2, ew3, eb3, sw1, sb1, sw2, sb2, sw3, sb3)


def kernel(hidden_states, gate_w, ew1, eb1, ew2, eb2, ew3, eb3, sw1, sb1,
           sw2, sb2, sw3, sb3):
    shape = hidden_states.shape
    x = hidden_states.reshape(-1, H)
    y = _run(x, gate_w,
             ew1, eb1.reshape(E, 1, I), ew2, eb2.reshape(E, 1, H),
             ew3, eb3.reshape(E, 1, I),
             sw1, sb1.reshape(1, SH_I), sw2, sb2.reshape(1, H),
             sw3, sb3.reshape(1, SH_I))
    return y.reshape(shape)
